# Initial kernel scaffold; baseline (speedup 1.0000x reference)
import jax
import jax.numpy as jnp
from jax.experimental import pallas as pl


def kernel(x, batch, weight, bias, mean_scale):
    raise NotImplementedError("write your pallas kernel here")


# speedup vs baseline: 6.3153x; 6.3153x over previous
import functools

import jax
import jax.numpy as jnp
from jax.experimental import pallas as pl

N = 100000
F = 128
G = 64
EPS = 1e-05
BLK = 2000
NBLK = N // BLK


def _stats_body(x_ref, b_ref, sums_ref, sq_ref, cnt_ref):
    j = pl.program_id(0)
    b = b_ref[0, 0, :]
    oh = (b[:, None] == jax.lax.broadcasted_iota(jnp.int32, (BLK, G), 1)).astype(
        jnp.float32
    )
    x = x_ref[...]
    s = jax.lax.dot_general(
        oh, x, (((0,), (0,)), ((), ())), precision=jax.lax.Precision.HIGHEST
    )
    q = jax.lax.dot_general(
        oh, x * x, (((0,), (0,)), ((), ())), precision=jax.lax.Precision.HIGHEST
    )
    c = jnp.sum(oh, axis=0)[None, :]

    @pl.when(j == 0)
    def _init():
        sums_ref[...] = s
        sq_ref[...] = q
        cnt_ref[...] = jnp.broadcast_to(c, (8, G))

    @pl.when(j != 0)
    def _acc():
        sums_ref[...] += s
        sq_ref[...] += q
        cnt_ref[...] += jnp.broadcast_to(c, (8, G))


def _apply_body(x_ref, b_ref, sums_ref, sq_ref, cnt_ref, w_ref, bias_ref, ms_ref,
                y_ref):
    c = jnp.maximum(cnt_ref[0, :], 1.0)[:, None]
    inv_c = 1.0 / c
    m = sums_ref[...] * inv_c
    q = sq_ref[...] * inv_c
    ms = ms_ref[0, :][None, :]
    var = q - m * m * ms * (2.0 - ms)
    a_tab = w_ref[0, :][None, :] * jax.lax.rsqrt(var + EPS)
    b_tab = bias_ref[0, :][None, :] - a_tab * m * ms

    b = b_ref[0, 0, :]
    oh = (b[:, None] == jax.lax.broadcasted_iota(jnp.int32, (BLK, G), 1)).astype(
        jnp.float32
    )
    arow = jax.lax.dot_general(
        oh, a_tab, (((1,), (0,)), ((), ())), precision=jax.lax.Precision.HIGHEST
    )
    brow = jax.lax.dot_general(
        oh, b_tab, (((1,), (0,)), ((), ())), precision=jax.lax.Precision.HIGHEST
    )
    y_ref[...] = arow * x_ref[...] + brow


@jax.jit
def kernel(x, batch, weight, bias, mean_scale):
    b3 = batch.astype(jnp.int32).reshape(NBLK, 1, BLK)
    w2 = weight.reshape(1, F)
    bias2 = bias.reshape(1, F)
    ms2 = mean_scale.reshape(1, F)

    sums, sq, cnt = pl.pallas_call(
        _stats_body,
        grid=(NBLK,),
        in_specs=[
            pl.BlockSpec((BLK, F), lambda j: (j, 0)),
            pl.BlockSpec((1, 1, BLK), lambda j: (j, 0, 0)),
        ],
        out_specs=[
            pl.BlockSpec((G, F), lambda j: (0, 0)),
            pl.BlockSpec((G, F), lambda j: (0, 0)),
            pl.BlockSpec((8, G), lambda j: (0, 0)),
        ],
        out_shape=[
            jax.ShapeDtypeStruct((G, F), jnp.float32),
            jax.ShapeDtypeStruct((G, F), jnp.float32),
            jax.ShapeDtypeStruct((8, G), jnp.float32),
        ],
    )(x, b3)

    y = pl.pallas_call(
        _apply_body,
        grid=(NBLK,),
        in_specs=[
            pl.BlockSpec((BLK, F), lambda j: (j, 0)),
            pl.BlockSpec((1, 1, BLK), lambda j: (j, 0, 0)),
            pl.BlockSpec((G, F), lambda j: (0, 0)),
            pl.BlockSpec((G, F), lambda j: (0, 0)),
            pl.BlockSpec((8, G), lambda j: (0, 0)),
            pl.BlockSpec((1, F), lambda j: (0, 0)),
            pl.BlockSpec((1, F), lambda j: (0, 0)),
            pl.BlockSpec((1, F), lambda j: (0, 0)),
        ],
        out_specs=pl.BlockSpec((BLK, F), lambda j: (j, 0)),
        out_shape=jax.ShapeDtypeStruct((N, F), jnp.float32),
    )(x, b3, sums, sq, cnt, w2, bias2, ms2)
    return y
